# baseline (speedup 1.0000x reference)
import functools

import jax
import jax.numpy as jnp
from jax import lax
from jax.experimental import pallas as pl
from jax.experimental.pallas import tpu as pltpu
from jax.experimental.pallas import tpu_sc as plsc

_B, _L = 4096, 200
_V, _D = 128 * 128, 32
_N = _B * _L

_NC, _NS, _LANES = 2, 16, 16
_NW = _NC * _NS
_RPW = _N // _NW
_CH = 1024
_NCH = _RPW // _CH
_G = _CH // 128


def _table_transform(emb_table, w2, b2):
    def body(emb_ref, w_ref, b_ref, out_ref):
        out_ref[...] = (
            jnp.dot(emb_ref[...], w_ref[...], preferred_element_type=jnp.float32,
                    precision=jax.lax.Precision.HIGHEST)
            + b_ref[...]
        )

    return pl.pallas_call(
        body,
        out_shape=jax.ShapeDtypeStruct((_V, _D), jnp.float32),
    )(emb_table, w2, b2)


def _tanh16(y):
    t = jnp.exp(y * 2.0)
    return 1.0 - 2.0 / (t + 1.0)


def _sc_lookup(table, idx2d, ll, w01):
    mesh = plsc.VectorSubcoreMesh(core_axis_name="c", subcore_axis_name="s")

    @functools.partial(
        pl.kernel,
        mesh=mesh,
        out_type=jax.ShapeDtypeStruct((_N, _D), jnp.float32),
        scratch_types=[
            pltpu.VMEM((2, _G, 128), jnp.int32),
            pltpu.VMEM((2, _CH, 2), jnp.float32),
            pltpu.VMEM((2, _CH, _D), jnp.float32),
            pltpu.VMEM((2, _D), jnp.float32),
            pltpu.SemaphoreType.DMA,
            pltpu.SemaphoreType.DMA,
            pltpu.SemaphoreType.DMA,
            pltpu.SemaphoreType.DMA,
            pltpu.SemaphoreType.DMA,
            pltpu.SemaphoreType.DMA,
            pltpu.SemaphoreType.DMA,
        ],
        compiler_params=pltpu.CompilerParams(
            needs_layout_passes=False, use_tc_tiling_on_sc=False
        ),
    )
    def k(table_hbm, idx_hbm, ll_hbm, w01_hbm, out_hbm,
          idx_v, ll_v, rows_v, w01_v,
          sem_idx, sem_ll0, sem_ll1, sem_g0, sem_g1, sem_o0, sem_o1):
        sem_ll = (sem_ll0, sem_ll1)
        sem_g = (sem_g0, sem_g1)
        sem_o = (sem_o0, sem_o1)
        wid = lax.axis_index("s") * _NC + lax.axis_index("c")
        base = wid * _RPW

        pltpu.sync_copy(w01_hbm, w01_v)
        w0a = w01_v[0, pl.ds(0, _LANES)]
        w0b = w01_v[0, pl.ds(_LANES, _LANES)]
        w1a = w01_v[1, pl.ds(0, _LANES)]
        w1b = w01_v[1, pl.ds(_LANES, _LANES)]
        zero16 = jnp.zeros((_LANES,), jnp.int32)
        one16 = jnp.full((_LANES,), 1, jnp.int32)

        def idx_cp(c, s):
            r0 = pl.multiple_of(base + c * _CH, _CH)
            i0 = pl.multiple_of(r0 // 128, _G)
            return pltpu.make_async_copy(
                idx_hbm.at[pl.ds(i0, _G), :], idx_v.at[s], sem_idx)

        def ll_cp(c, s):
            r0 = pl.multiple_of(base + c * _CH, _CH)
            return pltpu.make_async_copy(
                ll_hbm.at[pl.ds(r0, _CH), :], ll_v.at[s], sem_ll[s])

        def gather_cp(s, j):
            return pltpu.make_async_copy(
                table_hbm.at[idx_v.at[s, j]],
                rows_v.at[s, pl.ds(j * 128, 128), :],
                sem_g[s])

        def out_cp(c, s):
            r0 = pl.multiple_of(base + c * _CH, _CH)
            return pltpu.make_async_copy(
                rows_v.at[s], out_hbm.at[pl.ds(r0, _CH), :], sem_o[s])

        def compute_chunk(s):
            def row_body(r, rcarry):
                rv = jnp.full((_LANES,), r, jnp.int32)
                lng = plsc.load_gather(ll_v.at[s], [rv, zero16])
                lat = plsc.load_gather(ll_v.at[s], [rv, one16])
                g0 = rows_v[s, r, pl.ds(0, _LANES)]
                g1 = rows_v[s, r, pl.ds(_LANES, _LANES)]
                y0 = g0 + lng * w0a + lat * w1a
                y1 = g1 + lng * w0b + lat * w1b
                rows_v[s, r, pl.ds(0, _LANES)] = _tanh16(y0)
                rows_v[s, r, pl.ds(_LANES, _LANES)] = _tanh16(y1)
                return rcarry

            lax.fori_loop(0, _CH, row_body, 0, unroll=2)

        idx_cp(0, 0).start()
        ll_cp(0, 0).start()
        idx_cp(0, 0).wait()
        for j in range(_G):
            gather_cp(0, j).start()

        def outer(p, carry):
            for s in (0, 1):
                c = 2 * p + s
                t = 1 - s
                nc = c + 1

                @pl.when(nc < _NCH)
                def _prefetch():
                    idx_cp(nc, t).start()
                    ll_cp(nc, t).start()
                    idx_cp(nc, t).wait()

                    @pl.when(c >= 1)
                    def _wb_done():
                        out_cp(c - 1, t).wait()

                    for j in range(_G):
                        gather_cp(t, j).start()

                @pl.when(c < _NCH)
                def _work():
                    for j in range(_G):
                        gather_cp(s, j).wait()
                    ll_cp(c, s).wait()
                    compute_chunk(s)
                    out_cp(c, s).start()

            return carry

        lax.fori_loop(0, (_NCH + 2) // 2, outer, 0)
        out_cp(_NCH - 2, (_NCH - 2) % 2).wait()
        out_cp(_NCH - 1, (_NCH - 1) % 2).wait()

    return k(table, idx2d, ll, w01)


def kernel(lngs, lats, grid_id, emb_table, W, b):
    table = _table_transform(emb_table, W[2:], b.reshape(1, _D))
    idx2d = grid_id.reshape(_N // 128, 128).astype(jnp.int32)
    ll = jnp.stack([lngs.reshape(_N), lats.reshape(_N)], axis=1)
    out = _sc_lookup(table, idx2d, ll, W[:2])
    return out.reshape(_B, _L, _D)


# speedup vs baseline: 2.1166x; 2.1166x over previous
import functools

import jax
import jax.numpy as jnp
from jax import lax
from jax.experimental import pallas as pl
from jax.experimental.pallas import tpu as pltpu
from jax.experimental.pallas import tpu_sc as plsc

_B, _L = 4096, 200
_V, _D = 128 * 128, 32
_N = _B * _L

_NC, _NS, _LANES = 2, 16, 16
_NW = _NC * _NS
_RPW = _N // _NW
_CH = 1024
_NCH = _RPW // _CH
_G = _CH // 128


def _table_transform(emb_table, w2, b2):
    def body(emb_ref, w_ref, b_ref, out_ref):
        out_ref[...] = (
            jnp.dot(emb_ref[...], w_ref[...], preferred_element_type=jnp.float32,
                    precision=jax.lax.Precision.HIGHEST)
            + b_ref[...]
        )

    return pl.pallas_call(
        body,
        out_shape=jax.ShapeDtypeStruct((_V, _D), jnp.float32),
    )(emb_table, w2, b2)


def _tanh16(y):
    t = jnp.exp(y * 2.0)
    return 1.0 - 2.0 / (t + 1.0)


def _sc_lookup(table, idx2d, lng_f, lat_f, w01):
    mesh = plsc.VectorSubcoreMesh(core_axis_name="c", subcore_axis_name="s")

    @functools.partial(
        pl.kernel,
        mesh=mesh,
        out_type=jax.ShapeDtypeStruct((_N, _D), jnp.float32),
        scratch_types=[
            pltpu.VMEM((2, _G, 128), jnp.int32),
            pltpu.VMEM((2, 2, _CH), jnp.float32),
            pltpu.VMEM((2, _CH, _D), jnp.float32),
            pltpu.VMEM((2, _D), jnp.float32),
            pltpu.SemaphoreType.DMA,
            pltpu.SemaphoreType.DMA,
            pltpu.SemaphoreType.DMA,
            pltpu.SemaphoreType.DMA,
            pltpu.SemaphoreType.DMA,
            pltpu.SemaphoreType.DMA,
            pltpu.SemaphoreType.DMA,
        ],
        compiler_params=pltpu.CompilerParams(
            needs_layout_passes=False, use_tc_tiling_on_sc=False
        ),
    )
    def k(table_hbm, idx_hbm, lng_hbm, lat_hbm, w01_hbm, out_hbm,
          idx_v, ll_v, rows_v, w01_v,
          sem_idx, sem_ll0, sem_ll1, sem_g0, sem_g1, sem_o0, sem_o1):
        sem_ll = (sem_ll0, sem_ll1)
        sem_g = (sem_g0, sem_g1)
        sem_o = (sem_o0, sem_o1)
        wid = lax.axis_index("s") * _NC + lax.axis_index("c")
        base = wid * _RPW

        pltpu.sync_copy(w01_hbm, w01_v)
        w0a = w01_v[0, pl.ds(0, _LANES)]
        w0b = w01_v[0, pl.ds(_LANES, _LANES)]
        w1a = w01_v[1, pl.ds(0, _LANES)]
        w1b = w01_v[1, pl.ds(_LANES, _LANES)]
        zero16 = jnp.zeros((_LANES,), jnp.int32)
        one16 = jnp.full((_LANES,), 1, jnp.int32)

        def idx_cp(c, s):
            r0 = pl.multiple_of(base + c * _CH, _CH)
            i0 = pl.multiple_of(r0 // 128, _G)
            return pltpu.make_async_copy(
                idx_hbm.at[pl.ds(i0, _G), :], idx_v.at[s], sem_idx)

        def ll_cps(c, s):
            r0 = pl.multiple_of(base + c * _CH, _CH)
            return (
                pltpu.make_async_copy(
                    lng_hbm.at[pl.ds(r0, _CH)], ll_v.at[s, 0], sem_ll[s]),
                pltpu.make_async_copy(
                    lat_hbm.at[pl.ds(r0, _CH)], ll_v.at[s, 1], sem_ll[s]),
            )

        def gather_cp(s, j):
            return pltpu.make_async_copy(
                table_hbm.at[idx_v.at[s, j]],
                rows_v.at[s, pl.ds(j * 128, 128), :],
                sem_g[s])

        def out_cp(c, s):
            r0 = pl.multiple_of(base + c * _CH, _CH)
            return pltpu.make_async_copy(
                rows_v.at[s], out_hbm.at[pl.ds(r0, _CH), :], sem_o[s])

        def compute_chunk(s):
            def row_body(r, rcarry):
                rv = jnp.full((_LANES,), r, jnp.int32)
                lng = plsc.load_gather(ll_v.at[s, 0], [rv])
                lat = plsc.load_gather(ll_v.at[s, 1], [rv])
                g0 = rows_v[s, r, pl.ds(0, _LANES)]
                g1 = rows_v[s, r, pl.ds(_LANES, _LANES)]
                y0 = g0 + lng * w0a + lat * w1a
                y1 = g1 + lng * w0b + lat * w1b
                rows_v[s, r, pl.ds(0, _LANES)] = _tanh16(y0)
                rows_v[s, r, pl.ds(_LANES, _LANES)] = _tanh16(y1)
                return rcarry

            lax.fori_loop(0, _CH, row_body, 0, unroll=2)

        idx_cp(0, 0).start()
        for cp in ll_cps(0, 0):
            cp.start()
        idx_cp(0, 0).wait()
        for j in range(_G):
            gather_cp(0, j).start()

        def outer(p, carry):
            for s in (0, 1):
                c = 2 * p + s
                t = 1 - s
                nc = c + 1

                @pl.when(nc < _NCH)
                def _prefetch():
                    idx_cp(nc, t).start()
                    for cp in ll_cps(nc, t):
                        cp.start()
                    idx_cp(nc, t).wait()

                    @pl.when(c >= 1)
                    def _wb_done():
                        out_cp(c - 1, t).wait()

                    for j in range(_G):
                        gather_cp(t, j).start()

                @pl.when(c < _NCH)
                def _work():
                    for j in range(_G):
                        gather_cp(s, j).wait()
                    for cp in ll_cps(c, s):
                        cp.wait()
                    compute_chunk(s)
                    out_cp(c, s).start()

            return carry

        lax.fori_loop(0, (_NCH + 2) // 2, outer, 0)
        out_cp(_NCH - 2, (_NCH - 2) % 2).wait()
        out_cp(_NCH - 1, (_NCH - 1) % 2).wait()

    return k(table, idx2d, lng_f, lat_f, w01)


def kernel(lngs, lats, grid_id, emb_table, W, b):
    table = _table_transform(emb_table, W[2:], b.reshape(1, _D))
    idx2d = grid_id.reshape(_N // 128, 128).astype(jnp.int32)
    out = _sc_lookup(table, idx2d, lngs.reshape(_N), lats.reshape(_N), W[:2])
    return out.reshape(_B, _L, _D)
